# baseline (device time: 32622 ns/iter reference)
import jax
import jax.numpy as jnp
from jax import lax
from jax.experimental import pallas as pl
from jax.experimental.pallas import tpu as pltpu

N_DEV = 8
BLOCK_M = 512
NBUF = 4


def kernel(x, dy, gamma):
    m_per, d = x.shape
    nchunks = m_per // BLOCK_M

    def body(x_hbm, dy_hbm, out_ref, acc_ref, xbuf, dybuf, copy_sems,
             comm_ref, send_sems, recv_sems):
        my = lax.axis_index("i")

        barrier_sem = pltpu.get_barrier_semaphore()
        for k in range(1, N_DEV):
            peer = lax.rem(my + k, N_DEV)
            pl.semaphore_signal(
                barrier_sem, inc=1,
                device_id=(peer,), device_id_type=pl.DeviceIdType.MESH,
            )
        pl.semaphore_wait(barrier_sem, N_DEV - 1)

        def x_copy(c):
            return pltpu.make_async_copy(
                x_hbm.at[pl.ds(c * BLOCK_M, BLOCK_M), :],
                xbuf.at[c % NBUF],
                copy_sems.at[c % NBUF, 0],
            )

        def dy_copy(c):
            return pltpu.make_async_copy(
                dy_hbm.at[pl.ds(c * BLOCK_M, BLOCK_M), :],
                dybuf.at[c % NBUF],
                copy_sems.at[c % NBUF, 1],
            )

        for c in range(min(NBUF, nchunks)):
            x_copy(c).start()
            dy_copy(c).start()

        acc_ref[...] = jnp.zeros_like(acc_ref)
        for c in range(nchunks):
            x_copy(c).wait()
            dy_copy(c).wait()
            xv = xbuf[c % NBUF]
            dyv = dybuf[c % NBUF]
            mu = jnp.mean(xv, axis=1, keepdims=True)
            xc = xv - mu
            var = jnp.mean(xc * xc, axis=1, keepdims=True)
            rstd = lax.rsqrt(var + 1e-5)
            xhat = xc * rstd
            dgamma_p = jnp.sum(dyv * xhat, axis=0, keepdims=True)
            dbeta_p = jnp.sum(dyv, axis=0, keepdims=True)
            acc_ref[...] += jnp.concatenate([dgamma_p, dbeta_p], axis=0)
            if c + NBUF < nchunks:
                x_copy(c + NBUF).start()
                dy_copy(c + NBUF).start()

        def desc(k):
            tgt = lax.rem(my + k, N_DEV)
            return pltpu.make_async_remote_copy(
                src_ref=acc_ref,
                dst_ref=comm_ref.at[k - 1],
                send_sem=send_sems.at[k - 1],
                recv_sem=recv_sems.at[k - 1],
                device_id=(tgt,),
                device_id_type=pl.DeviceIdType.MESH,
            )

        for k in range(1, N_DEV):
            desc(k).start()

        total = acc_ref[...]
        for k in range(1, N_DEV):
            desc(k).wait_recv()
            total = total + comm_ref[k - 1]
        out_ref[...] = total

        for k in range(1, N_DEV):
            desc(k).wait_send()

    return pl.pallas_call(
        body,
        out_shape=jax.ShapeDtypeStruct((2, d), jnp.float32),
        in_specs=[
            pl.BlockSpec(memory_space=pltpu.MemorySpace.HBM),
            pl.BlockSpec(memory_space=pltpu.MemorySpace.HBM),
        ],
        out_specs=pl.BlockSpec(memory_space=pltpu.VMEM),
        scratch_shapes=[
            pltpu.VMEM((2, d), jnp.float32),
            pltpu.VMEM((NBUF, BLOCK_M, d), jnp.float32),
            pltpu.VMEM((NBUF, BLOCK_M, d), jnp.float32),
            pltpu.SemaphoreType.DMA((NBUF, 2)),
            pltpu.VMEM((N_DEV - 1, 2, d), jnp.float32),
            pltpu.SemaphoreType.DMA((N_DEV - 1,)),
            pltpu.SemaphoreType.DMA((N_DEV - 1,)),
        ],
        compiler_params=pltpu.CompilerParams(
            collective_id=0,
            vmem_limit_bytes=60 * 1024 * 1024,
        ),
    )(x, dy)
